# baseline (speedup 1.0000x reference)
import functools

import jax
import jax.numpy as jnp
from jax import lax
from jax.experimental import pallas as pl
from jax.experimental.pallas import tpu as pltpu
from jax.experimental.pallas import tpu_sc as plsc

_N_SITES = 10000
_N_EDGES = 320000
_N_GRAPHS = 64
_SITE_LEN = 256
_BOND_LEN = 16
_STATE_LEN = 128
_H1 = 512
_H2 = 512

_NC = 2
_NS = 16
_NW = _NC * _NS

_ROW_TILE = 400
_N_TILES = _N_SITES // _ROW_TILE
_ACC_ROWS = 10240
_ROWS_PER_TEC = _ACC_ROWS // _NS

_CHUNK = 125
_CHUNKS_PER_DMA = 16
_DMA_EDGES = _CHUNK * _CHUNKS_PER_DMA
_EDGES_PER_WORKER = _N_EDGES // _NW
_DMAS_PER_WORKER = _EDGES_PER_WORKER // _DMA_EDGES
_GROUPS = _DMA_EDGES // 16


def _sc_scatter_body(bt_hbm, idx_hbm, sum_hbm, cnt_hbm,
                     idx_v, xbuf, rowbuf, ones_v, zbuf, acc_sum, acc_cnt):
    cid = lax.axis_index("c")
    tid = lax.axis_index("s")
    wid = tid * _NC + cid
    i32 = jnp.int32
    f32 = jnp.float32

    def _zero(i, _):
        zbuf[i] = jnp.zeros((16,), f32)
        return 0
    lax.fori_loop(0, _ROWS_PER_TEC, _zero, 0)

    def _one(i, _):
        ones_v[i] = jnp.ones((16,), f32)
        return 0
    lax.fori_loop(0, _CHUNK, _one, 0)

    pltpu.sync_copy(zbuf, acc_sum.at[pl.ds(tid * _ROWS_PER_TEC, _ROWS_PER_TEC)])
    pltpu.sync_copy(zbuf, acc_cnt.at[pl.ds(tid * _ROWS_PER_TEC, _ROWS_PER_TEC)])
    plsc.subcore_barrier()

    base = wid * _EDGES_PER_WORKER
    iota16 = lax.iota(i32, 16)
    diag = [jnp.bitwise_and(iota16 + d, _BOND_LEN - 1) for d in range(_BOND_LEN)]

    def _outer(o, _):
        pltpu.sync_copy(idx_hbm.at[wid, pl.ds(o * _CHUNKS_PER_DMA, _CHUNKS_PER_DMA)],
                        idx_v)
        pltpu.sync_copy(bt_hbm.at[:, pl.ds(base + o * _DMA_EDGES, _DMA_EDGES)],
                        xbuf)

        def _tr(g, _):
            edges = iota16 + g * 16
            for d in range(_BOND_LEN):
                vals = plsc.load_gather(xbuf, [diag[d], edges])
                plsc.store_scatter(rowbuf, [edges, diag[d]], vals)
            return 0
        lax.fori_loop(0, _GROUPS, _tr, 0)

        for j in range(_CHUNKS_PER_DMA):
            idx_row = idx_v.at[j]
            pltpu.sync_copy(rowbuf.at[pl.ds(j * _CHUNK, _CHUNK)],
                            acc_sum.at[idx_row], add=True)
            pltpu.sync_copy(ones_v, acc_cnt.at[idx_row], add=True)
        return 0
    lax.fori_loop(0, _DMAS_PER_WORKER, _outer, 0)

    plsc.subcore_barrier()
    sl = pl.ds(tid * _ROWS_PER_TEC, _ROWS_PER_TEC)
    pltpu.sync_copy(acc_sum.at[sl], sum_hbm.at[cid, sl])
    pltpu.sync_copy(acc_cnt.at[sl], cnt_hbm.at[cid, sl])


@functools.lru_cache(maxsize=None)
def _get_sc_scatter():
  return pl.kernel(
    _sc_scatter_body,
    out_type=(
        jax.ShapeDtypeStruct((_NC, _ACC_ROWS, _BOND_LEN), jnp.float32),
        jax.ShapeDtypeStruct((_NC, _ACC_ROWS, _BOND_LEN), jnp.float32),
    ),
    mesh=plsc.VectorSubcoreMesh(core_axis_name="c", subcore_axis_name="s",
                                num_cores=_NC, num_subcores=_NS),
    compiler_params=pltpu.CompilerParams(use_tc_tiling_on_sc=False,
                                         needs_layout_passes=False),
    scratch_types=[
        pltpu.VMEM((_CHUNKS_PER_DMA, _CHUNK), jnp.int32),
        pltpu.VMEM((_BOND_LEN, _DMA_EDGES), jnp.float32),
        pltpu.VMEM((_DMA_EDGES, _BOND_LEN), jnp.float32),
        pltpu.VMEM((_CHUNK, _BOND_LEN), jnp.float32),
        pltpu.VMEM((_ROWS_PER_TEC, _BOND_LEN), jnp.float32),
        pltpu.VMEM_SHARED((_ACC_ROWS, _BOND_LEN), jnp.float32),
        pltpu.VMEM_SHARED((_ACC_ROWS, _BOND_LEN), jnp.float32),
    ],
  )


def _tc_mlp_body(sum_ref, cnt_ref, sites_ref, gts_ref, states_ref,
                 w1a_ref, w1b_ref, w1c_ref, b1_ref,
                 w2_ref, b2_ref, w3_ref, b3_ref, out_ref):
    f32 = jnp.float32
    seg_sum = sum_ref[0] + sum_ref[1]
    seg_cnt = cnt_ref[0] + cnt_ref[1]
    bp = seg_sum / jnp.maximum(seg_cnt, 1.0)

    g = gts_ref[0, 0, :]
    iota = lax.broadcasted_iota(jnp.int32, (_ROW_TILE, _N_GRAPHS), 1)
    onehot = (g[:, None] == iota).astype(f32)
    sg = jnp.dot(onehot, states_ref[...], preferred_element_type=f32)

    h = (jnp.dot(bp, w1a_ref[...], preferred_element_type=f32)
         + jnp.dot(sites_ref[...], w1b_ref[...], preferred_element_type=f32)
         + jnp.dot(sg, w1c_ref[...], preferred_element_type=f32)
         + b1_ref[...])
    h = jnp.maximum(h, 0.0)
    h = jnp.maximum(jnp.dot(h, w2_ref[...], preferred_element_type=f32)
                    + b2_ref[...], 0.0)
    out_ref[...] = jnp.maximum(jnp.dot(h, w3_ref[...], preferred_element_type=f32)
                               + b3_ref[...], 0.0)


_tc_mlp = pl.pallas_call(
    _tc_mlp_body,
    grid=(_N_TILES,),
    in_specs=[
        pl.BlockSpec((_NC, _ROW_TILE, _BOND_LEN), lambda i: (0, i, 0)),
        pl.BlockSpec((_NC, _ROW_TILE, _BOND_LEN), lambda i: (0, i, 0)),
        pl.BlockSpec((_ROW_TILE, _SITE_LEN), lambda i: (i, 0)),
        pl.BlockSpec((1, 1, _ROW_TILE), lambda i: (i, 0, 0)),
        pl.BlockSpec((_N_GRAPHS, _STATE_LEN), lambda i: (0, 0)),
        pl.BlockSpec((_BOND_LEN, _H1), lambda i: (0, 0)),
        pl.BlockSpec((_SITE_LEN, _H1), lambda i: (0, 0)),
        pl.BlockSpec((_STATE_LEN, _H1), lambda i: (0, 0)),
        pl.BlockSpec((1, _H1), lambda i: (0, 0)),
        pl.BlockSpec((_H1, _H2), lambda i: (0, 0)),
        pl.BlockSpec((1, _H2), lambda i: (0, 0)),
        pl.BlockSpec((_H2, _SITE_LEN), lambda i: (0, 0)),
        pl.BlockSpec((1, _SITE_LEN), lambda i: (0, 0)),
    ],
    out_specs=pl.BlockSpec((_ROW_TILE, _SITE_LEN), lambda i: (i, 0)),
    out_shape=jax.ShapeDtypeStruct((_N_SITES, _SITE_LEN), jnp.float32),
)


@jax.jit
def kernel(sites, bonds, states, W1, b1, W2, b2, W3, b3, indices1,
           graph_to_sites):
    i32 = jnp.int32

    idx_p = indices1.astype(i32).reshape(
        _NW, _EDGES_PER_WORKER // _CHUNK, _CHUNK)

    seg_sum, seg_cnt = _get_sc_scatter()(bonds.T, idx_p)

    gts_p = graph_to_sites.astype(i32).reshape(_N_TILES, 1, _ROW_TILE)

    out = _tc_mlp(seg_sum, seg_cnt, sites, gts_p, states,
                  W1[:_BOND_LEN], W1[_BOND_LEN:_BOND_LEN + _SITE_LEN],
                  W1[_BOND_LEN + _SITE_LEN:], b1.reshape(1, _H1),
                  W2, b2.reshape(1, _H2), W3, b3.reshape(1, _SITE_LEN))
    return out


# speedup vs baseline: 1.1313x; 1.1313x over previous
import functools

import jax
import jax.numpy as jnp
from jax import lax
from jax.experimental import pallas as pl
from jax.experimental.pallas import tpu as pltpu
from jax.experimental.pallas import tpu_sc as plsc

_N_SITES = 10000
_N_EDGES = 320000
_N_GRAPHS = 64
_SITE_LEN = 256
_BOND_LEN = 16
_STATE_LEN = 128
_H1 = 512
_H2 = 512

_NC = 2
_NS = 16
_NW = _NC * _NS

_ROW_TILE = 400
_N_TILES = _N_SITES // _ROW_TILE
_ACC_ROWS = 10240
_ROWS_PER_TEC = _ACC_ROWS // _NS

_CHUNK = 125
_CHUNKS_PER_DMA = 16
_DMA_EDGES = _CHUNK * _CHUNKS_PER_DMA
_EDGES_PER_WORKER = _N_EDGES // _NW
_DMAS_PER_WORKER = _EDGES_PER_WORKER // _DMA_EDGES
_GROUPS = _DMA_EDGES // 16


def _sc_scatter_body(bt_hbm, idx_hbm, sum_hbm, cnt_hbm,
                     idx_va, idx_vb, xbuf, rowbuf_a, rowbuf_b, ones_v, zbuf,
                     scat_sem, acc_sum, acc_cnt):
    cid = lax.axis_index("c")
    tid = lax.axis_index("s")
    wid = tid * _NC + cid
    i32 = jnp.int32
    f32 = jnp.float32

    def _zero(i, _):
        zbuf[i] = jnp.zeros((16,), f32)
        return 0
    lax.fori_loop(0, 128, _zero, 0)

    def _one(i, _):
        ones_v[i] = jnp.ones((16,), f32)
        return 0
    lax.fori_loop(0, _CHUNK, _one, 0)

    for k in range(_ROWS_PER_TEC // 128):
        sl0 = pl.ds(tid * _ROWS_PER_TEC + k * 128, 128)
        pltpu.sync_copy(zbuf, acc_sum.at[sl0])
        pltpu.sync_copy(zbuf, acc_cnt.at[sl0])
    plsc.subcore_barrier()

    base = wid * _EDGES_PER_WORKER
    iota16 = lax.iota(i32, 16)
    cols = [jnp.full((16,), f, i32) for f in range(_BOND_LEN)]

    descs = {}
    for o in range(_DMAS_PER_WORKER):
        idx_v = idx_va if o % 2 == 0 else idx_vb
        rowbuf = rowbuf_a if o % 2 == 0 else rowbuf_b
        if o >= 2:
            for d in descs.pop(o - 2):
                d.wait()

        pltpu.sync_copy(idx_hbm.at[wid, pl.ds(o * _CHUNKS_PER_DMA, _CHUNKS_PER_DMA)],
                        idx_v)
        pltpu.sync_copy(bt_hbm.at[:, pl.ds(base + o * _DMA_EDGES, _DMA_EDGES)],
                        xbuf)

        def _tr(g, _, xbuf=xbuf, rowbuf=rowbuf):
            rows = iota16 + g * 16
            for f in range(_BOND_LEN):
                vals = xbuf[f, pl.ds(g * 16, 16)]
                plsc.store_scatter(rowbuf, [rows, cols[f]], vals)
            return 0
        lax.fori_loop(0, _GROUPS, _tr, 0)

        ds = []
        for j in range(_CHUNKS_PER_DMA):
            idx_row = idx_v.at[j]
            ds.append(pltpu.async_copy(rowbuf.at[pl.ds(j * _CHUNK, _CHUNK)],
                                       acc_sum.at[idx_row], scat_sem, add=True))
            ds.append(pltpu.async_copy(ones_v, acc_cnt.at[idx_row], scat_sem,
                                       add=True))
        descs[o] = ds
    for o in sorted(descs):
        for d in descs[o]:
            d.wait()

    plsc.subcore_barrier()
    sl = pl.ds(tid * _ROWS_PER_TEC, _ROWS_PER_TEC)
    pltpu.sync_copy(acc_sum.at[sl], sum_hbm.at[cid, sl])
    pltpu.sync_copy(acc_cnt.at[sl], cnt_hbm.at[cid, sl])


@functools.lru_cache(maxsize=None)
def _get_sc_scatter():
  return pl.kernel(
    _sc_scatter_body,
    out_type=(
        jax.ShapeDtypeStruct((_NC, _ACC_ROWS, _BOND_LEN), jnp.float32),
        jax.ShapeDtypeStruct((_NC, _ACC_ROWS, _BOND_LEN), jnp.float32),
    ),
    mesh=plsc.VectorSubcoreMesh(core_axis_name="c", subcore_axis_name="s",
                                num_cores=_NC, num_subcores=_NS),
    compiler_params=pltpu.CompilerParams(use_tc_tiling_on_sc=False,
                                         needs_layout_passes=False),
    scratch_types=[
        pltpu.VMEM((_CHUNKS_PER_DMA, _CHUNK), jnp.int32),
        pltpu.VMEM((_CHUNKS_PER_DMA, _CHUNK), jnp.int32),
        pltpu.VMEM((_BOND_LEN, _DMA_EDGES), jnp.float32),
        pltpu.VMEM((_DMA_EDGES, _BOND_LEN), jnp.float32),
        pltpu.VMEM((_DMA_EDGES, _BOND_LEN), jnp.float32),
        pltpu.VMEM((_CHUNK, _BOND_LEN), jnp.float32),
        pltpu.VMEM((128, _BOND_LEN), jnp.float32),
        pltpu.SemaphoreType.DMA,
        pltpu.VMEM_SHARED((_ACC_ROWS, _BOND_LEN), jnp.float32),
        pltpu.VMEM_SHARED((_ACC_ROWS, _BOND_LEN), jnp.float32),
    ],
  )


def _tc_mlp_body(sum_ref, cnt_ref, sites_ref, gts_ref, states_ref,
                 w1a_ref, w1b_ref, w1c_ref, b1_ref,
                 w2_ref, b2_ref, w3_ref, b3_ref, out_ref):
    f32 = jnp.float32
    seg_sum = sum_ref[0] + sum_ref[1]
    seg_cnt = cnt_ref[0] + cnt_ref[1]
    bp = seg_sum / jnp.maximum(seg_cnt, 1.0)

    g = gts_ref[0, 0, :]
    iota = lax.broadcasted_iota(jnp.int32, (_ROW_TILE, _N_GRAPHS), 1)
    onehot = (g[:, None] == iota).astype(f32)
    sg = jnp.dot(onehot, states_ref[...], preferred_element_type=f32)

    h = (jnp.dot(bp, w1a_ref[...], preferred_element_type=f32)
         + jnp.dot(sites_ref[...], w1b_ref[...], preferred_element_type=f32)
         + jnp.dot(sg, w1c_ref[...], preferred_element_type=f32)
         + b1_ref[...])
    h = jnp.maximum(h, 0.0)
    h = jnp.maximum(jnp.dot(h, w2_ref[...], preferred_element_type=f32)
                    + b2_ref[...], 0.0)
    out_ref[...] = jnp.maximum(jnp.dot(h, w3_ref[...], preferred_element_type=f32)
                               + b3_ref[...], 0.0)


_tc_mlp = pl.pallas_call(
    _tc_mlp_body,
    grid=(_N_TILES,),
    in_specs=[
        pl.BlockSpec((_NC, _ROW_TILE, _BOND_LEN), lambda i: (0, i, 0)),
        pl.BlockSpec((_NC, _ROW_TILE, _BOND_LEN), lambda i: (0, i, 0)),
        pl.BlockSpec((_ROW_TILE, _SITE_LEN), lambda i: (i, 0)),
        pl.BlockSpec((1, 1, _ROW_TILE), lambda i: (i, 0, 0)),
        pl.BlockSpec((_N_GRAPHS, _STATE_LEN), lambda i: (0, 0)),
        pl.BlockSpec((_BOND_LEN, _H1), lambda i: (0, 0)),
        pl.BlockSpec((_SITE_LEN, _H1), lambda i: (0, 0)),
        pl.BlockSpec((_STATE_LEN, _H1), lambda i: (0, 0)),
        pl.BlockSpec((1, _H1), lambda i: (0, 0)),
        pl.BlockSpec((_H1, _H2), lambda i: (0, 0)),
        pl.BlockSpec((1, _H2), lambda i: (0, 0)),
        pl.BlockSpec((_H2, _SITE_LEN), lambda i: (0, 0)),
        pl.BlockSpec((1, _SITE_LEN), lambda i: (0, 0)),
    ],
    out_specs=pl.BlockSpec((_ROW_TILE, _SITE_LEN), lambda i: (i, 0)),
    out_shape=jax.ShapeDtypeStruct((_N_SITES, _SITE_LEN), jnp.float32),
)


@jax.jit
def kernel(sites, bonds, states, W1, b1, W2, b2, W3, b3, indices1,
           graph_to_sites):
    i32 = jnp.int32

    idx_p = indices1.astype(i32).reshape(
        _NW, _EDGES_PER_WORKER // _CHUNK, _CHUNK)

    seg_sum, seg_cnt = _get_sc_scatter()(bonds.T, idx_p)

    gts_p = graph_to_sites.astype(i32).reshape(_N_TILES, 1, _ROW_TILE)

    out = _tc_mlp(seg_sum, seg_cnt, sites, gts_p, states,
                  W1[:_BOND_LEN], W1[_BOND_LEN:_BOND_LEN + _SITE_LEN],
                  W1[_BOND_LEN + _SITE_LEN:], b1.reshape(1, _H1),
                  W2, b2.reshape(1, _H2), W3, b3.reshape(1, _SITE_LEN))
    return out


# speedup vs baseline: 1.1446x; 1.0117x over previous
import functools

import jax
import jax.numpy as jnp
from jax import lax
from jax.experimental import pallas as pl
from jax.experimental.pallas import tpu as pltpu
from jax.experimental.pallas import tpu_sc as plsc

_N_SITES = 10000
_N_EDGES = 320000
_N_GRAPHS = 64
_SITE_LEN = 256
_BOND_LEN = 16
_STATE_LEN = 128
_H1 = 512
_H2 = 512

_NC = 2
_NS = 16
_NW = _NC * _NS

_ROW_TILE = 400
_N_TILES = _N_SITES // _ROW_TILE
_ACC_ROWS = 10240
_ROWS_PER_TEC = _ACC_ROWS // _NS

_CHUNK = 125
_CHUNKS_PER_DMA = 16
_DMA_EDGES = _CHUNK * _CHUNKS_PER_DMA
_EDGES_PER_WORKER = _N_EDGES // _NW
_DMAS_PER_WORKER = _EDGES_PER_WORKER // _DMA_EDGES
_GROUPS = _DMA_EDGES // 16


def _sc_scatter_body(bt_hbm, idx_hbm, sum_hbm, cnt_hbm,
                     idx_va, idx_vb, xbuf, rowbuf_a, rowbuf_b, ones_v, zbuf,
                     scat_sem, acc_sum, acc_cnt):
    cid = lax.axis_index("c")
    tid = lax.axis_index("s")
    wid = tid * _NC + cid
    i32 = jnp.int32
    f32 = jnp.float32

    def _zero(i, _):
        zbuf[i] = jnp.zeros((16,), f32)
        return 0
    lax.fori_loop(0, 128, _zero, 0)

    def _one(i, _):
        ones_v[i] = jnp.ones((16,), f32)
        return 0
    lax.fori_loop(0, _CHUNK, _one, 0)

    for k in range(_ROWS_PER_TEC // 128):
        sl0 = pl.ds(tid * _ROWS_PER_TEC + k * 128, 128)
        pltpu.sync_copy(zbuf, acc_sum.at[sl0])
        pltpu.sync_copy(zbuf, acc_cnt.at[sl0])
    plsc.subcore_barrier()

    base = wid * _EDGES_PER_WORKER
    iota16 = lax.iota(i32, 16)
    cols = [jnp.full((16,), f, i32) for f in range(_BOND_LEN)]

    descs = {}
    for o in range(_DMAS_PER_WORKER):
        idx_v = idx_va if o % 2 == 0 else idx_vb
        rowbuf = rowbuf_a if o % 2 == 0 else rowbuf_b
        if o >= 2:
            for d in descs.pop(o - 2):
                d.wait()

        pltpu.sync_copy(idx_hbm.at[wid, pl.ds(o * _CHUNKS_PER_DMA, _CHUNKS_PER_DMA)],
                        idx_v)
        pltpu.sync_copy(bt_hbm.at[:, pl.ds(base + o * _DMA_EDGES, _DMA_EDGES)],
                        xbuf)

        def _tr(g, _, xbuf=xbuf, rowbuf=rowbuf):
            rows = iota16 + g * 16
            for f in range(_BOND_LEN):
                vals = xbuf[f, pl.ds(g * 16, 16)]
                plsc.store_scatter(rowbuf, [rows, cols[f]], vals)
            return 0
        lax.fori_loop(0, _GROUPS, _tr, 0)

        ds = []
        for j in range(_CHUNKS_PER_DMA):
            idx_row = idx_v.at[j]
            ds.append(pltpu.async_copy(rowbuf.at[pl.ds(j * _CHUNK, _CHUNK)],
                                       acc_sum.at[idx_row], scat_sem, add=True))
            ds.append(pltpu.async_copy(ones_v, acc_cnt.at[idx_row], scat_sem,
                                       add=True))
        descs[o] = ds
    for o in sorted(descs):
        for d in descs[o]:
            d.wait()

    plsc.subcore_barrier()
    sl = pl.ds(tid * _ROWS_PER_TEC, _ROWS_PER_TEC)
    pltpu.sync_copy(acc_sum.at[sl], sum_hbm.at[cid, sl])
    pltpu.sync_copy(acc_cnt.at[sl], cnt_hbm.at[cid, sl])


@functools.lru_cache(maxsize=None)
def _get_sc_scatter():
  return pl.kernel(
    _sc_scatter_body,
    out_type=(
        jax.ShapeDtypeStruct((_NC, _ACC_ROWS, _BOND_LEN), jnp.float32),
        jax.ShapeDtypeStruct((_NC, _ACC_ROWS, _BOND_LEN), jnp.float32),
    ),
    mesh=plsc.VectorSubcoreMesh(core_axis_name="c", subcore_axis_name="s",
                                num_cores=_NC, num_subcores=_NS),
    compiler_params=pltpu.CompilerParams(use_tc_tiling_on_sc=False,
                                         needs_layout_passes=False),
    scratch_types=[
        pltpu.VMEM((_CHUNKS_PER_DMA, _CHUNK), jnp.int32),
        pltpu.VMEM((_CHUNKS_PER_DMA, _CHUNK), jnp.int32),
        pltpu.VMEM((_BOND_LEN, _DMA_EDGES), jnp.float32),
        pltpu.VMEM((_DMA_EDGES, _BOND_LEN), jnp.float32),
        pltpu.VMEM((_DMA_EDGES, _BOND_LEN), jnp.float32),
        pltpu.VMEM((_CHUNK, _BOND_LEN), jnp.float32),
        pltpu.VMEM((128, _BOND_LEN), jnp.float32),
        pltpu.SemaphoreType.DMA,
        pltpu.VMEM_SHARED((_ACC_ROWS, _BOND_LEN), jnp.float32),
        pltpu.VMEM_SHARED((_ACC_ROWS, _BOND_LEN), jnp.float32),
    ],
  )


def _tc_pre_body(sites_ref, gts_ref, states_ref, w1b_ref, w1c_ref, b1_ref,
                 h1p_ref):
    f32 = jnp.float32
    g = gts_ref[0, 0, :]
    iota = lax.broadcasted_iota(jnp.int32, (_ROW_TILE, _N_GRAPHS), 1)
    onehot = (g[:, None] == iota).astype(f32)
    sg = jnp.dot(onehot, states_ref[...], preferred_element_type=f32)
    h1p_ref[...] = (jnp.dot(sites_ref[...], w1b_ref[...],
                            preferred_element_type=f32)
                    + jnp.dot(sg, w1c_ref[...], preferred_element_type=f32)
                    + b1_ref[...])


_tc_pre = pl.pallas_call(
    _tc_pre_body,
    grid=(_N_TILES,),
    in_specs=[
        pl.BlockSpec((_ROW_TILE, _SITE_LEN), lambda i: (i, 0)),
        pl.BlockSpec((1, 1, _ROW_TILE), lambda i: (i, 0, 0)),
        pl.BlockSpec((_N_GRAPHS, _STATE_LEN), lambda i: (0, 0)),
        pl.BlockSpec((_SITE_LEN, _H1), lambda i: (0, 0)),
        pl.BlockSpec((_STATE_LEN, _H1), lambda i: (0, 0)),
        pl.BlockSpec((1, _H1), lambda i: (0, 0)),
    ],
    out_specs=pl.BlockSpec((_ROW_TILE, _H1), lambda i: (i, 0)),
    out_shape=jax.ShapeDtypeStruct((_N_SITES, _H1), jnp.float32),
)


def _tc_post_body(sum_ref, cnt_ref, h1p_ref, w1a_ref,
                  w2_ref, b2_ref, w3_ref, b3_ref, out_ref):
    f32 = jnp.float32
    seg_sum = sum_ref[0] + sum_ref[1]
    seg_cnt = cnt_ref[0] + cnt_ref[1]
    bp = seg_sum / jnp.maximum(seg_cnt, 1.0)

    h = jnp.maximum(h1p_ref[...]
                    + jnp.dot(bp, w1a_ref[...], preferred_element_type=f32),
                    0.0)
    h = jnp.maximum(jnp.dot(h, w2_ref[...], preferred_element_type=f32)
                    + b2_ref[...], 0.0)
    out_ref[...] = jnp.maximum(jnp.dot(h, w3_ref[...], preferred_element_type=f32)
                               + b3_ref[...], 0.0)


_tc_post = pl.pallas_call(
    _tc_post_body,
    grid=(_N_TILES,),
    in_specs=[
        pl.BlockSpec((_NC, _ROW_TILE, _BOND_LEN), lambda i: (0, i, 0)),
        pl.BlockSpec((_NC, _ROW_TILE, _BOND_LEN), lambda i: (0, i, 0)),
        pl.BlockSpec((_ROW_TILE, _H1), lambda i: (i, 0)),
        pl.BlockSpec((_BOND_LEN, _H1), lambda i: (0, 0)),
        pl.BlockSpec((_H1, _H2), lambda i: (0, 0)),
        pl.BlockSpec((1, _H2), lambda i: (0, 0)),
        pl.BlockSpec((_H2, _SITE_LEN), lambda i: (0, 0)),
        pl.BlockSpec((1, _SITE_LEN), lambda i: (0, 0)),
    ],
    out_specs=pl.BlockSpec((_ROW_TILE, _SITE_LEN), lambda i: (i, 0)),
    out_shape=jax.ShapeDtypeStruct((_N_SITES, _SITE_LEN), jnp.float32),
)


@jax.jit
def kernel(sites, bonds, states, W1, b1, W2, b2, W3, b3, indices1,
           graph_to_sites):
    i32 = jnp.int32

    idx_p = indices1.astype(i32).reshape(
        _NW, _EDGES_PER_WORKER // _CHUNK, _CHUNK)

    seg_sum, seg_cnt = _get_sc_scatter()(bonds.T, idx_p)

    gts_p = graph_to_sites.astype(i32).reshape(_N_TILES, 1, _ROW_TILE)

    h1p = _tc_pre(sites, gts_p, states,
                  W1[_BOND_LEN:_BOND_LEN + _SITE_LEN],
                  W1[_BOND_LEN + _SITE_LEN:], b1.reshape(1, _H1))
    out = _tc_post(seg_sum, seg_cnt, h1p, W1[:_BOND_LEN],
                   W2, b2.reshape(1, _H2), W3, b3.reshape(1, _SITE_LEN))
    return out


# speedup vs baseline: 1.2220x; 1.0676x over previous
import functools

import jax
import jax.numpy as jnp
from jax import lax
from jax.experimental import pallas as pl
from jax.experimental.pallas import tpu as pltpu
from jax.experimental.pallas import tpu_sc as plsc

_N_SITES = 10000
_N_EDGES = 320000
_N_GRAPHS = 64
_SITE_LEN = 256
_BOND_LEN = 16
_STATE_LEN = 128
_H1 = 512
_H2 = 512

_NC = 2
_NS = 16
_NW = _NC * _NS

_ROW_TILE = 400
_N_TILES = _N_SITES // _ROW_TILE
_ACC_ROWS = 10240
_ROWS_PER_TEC = _ACC_ROWS // _NS

_CHUNK = 125
_CHUNKS_PER_DMA = 16
_DMA_EDGES = _CHUNK * _CHUNKS_PER_DMA
_EDGES_PER_WORKER = _N_EDGES // _NW
_DMAS_PER_WORKER = _EDGES_PER_WORKER // _DMA_EDGES
_GROUPS = _DMA_EDGES // 16


def _sc_scatter_body(bt_hbm, idx_hbm, sum_hbm, cnt_hbm,
                     idx_va, idx_vb, xbuf, rowbuf_a, rowbuf_b, ones_v, zbuf,
                     scat_sem, acc_sum, acc_cnt):
    cid = lax.axis_index("c")
    tid = lax.axis_index("s")
    wid = tid * _NC + cid
    i32 = jnp.int32
    f32 = jnp.float32

    def _zero(i, _):
        zbuf[i] = jnp.zeros((16,), f32)
        return 0
    lax.fori_loop(0, 128, _zero, 0)

    def _one(i, _):
        ones_v[i] = jnp.ones((16,), f32)
        return 0
    lax.fori_loop(0, _CHUNK, _one, 0)

    for k in range(_ROWS_PER_TEC // 128):
        sl0 = pl.ds(tid * _ROWS_PER_TEC + k * 128, 128)
        pltpu.sync_copy(zbuf, acc_sum.at[sl0])
        pltpu.sync_copy(zbuf, acc_cnt.at[sl0])
    plsc.subcore_barrier()

    base = wid * _EDGES_PER_WORKER
    iota16 = lax.iota(i32, 16)
    cols = [jnp.full((16,), f, i32) for f in range(_BOND_LEN)]

    descs = {}
    for o in range(_DMAS_PER_WORKER):
        idx_v = idx_va if o % 2 == 0 else idx_vb
        rowbuf = rowbuf_a if o % 2 == 0 else rowbuf_b
        if o >= 2:
            for d in descs.pop(o - 2):
                d.wait()

        pltpu.sync_copy(idx_hbm.at[wid, pl.ds(o * _CHUNKS_PER_DMA, _CHUNKS_PER_DMA)],
                        idx_v)
        pltpu.sync_copy(bt_hbm.at[:, pl.ds(base + o * _DMA_EDGES, _DMA_EDGES)],
                        xbuf)

        def _tr(g, _, xbuf=xbuf, rowbuf=rowbuf):
            rows = iota16 + g * 16
            for f in range(_BOND_LEN):
                vals = xbuf[f, pl.ds(g * 16, 16)]
                plsc.store_scatter(rowbuf, [rows, cols[f]], vals)
            return 0
        lax.fori_loop(0, _GROUPS, _tr, 0)

        ds = []
        for j in range(_CHUNKS_PER_DMA):
            idx_row = idx_v.at[j]
            ds.append(pltpu.async_copy(rowbuf.at[pl.ds(j * _CHUNK, _CHUNK)],
                                       acc_sum.at[idx_row], scat_sem, add=True))
            ds.append(pltpu.async_copy(ones_v, acc_cnt.at[idx_row], scat_sem,
                                       add=True))
        descs[o] = ds
    for o in sorted(descs):
        for d in descs[o]:
            d.wait()

    plsc.subcore_barrier()
    sl = pl.ds(tid * _ROWS_PER_TEC, _ROWS_PER_TEC)
    pltpu.sync_copy(acc_sum.at[sl], sum_hbm.at[cid, sl])
    pltpu.sync_copy(acc_cnt.at[sl], cnt_hbm.at[cid, sl])


@functools.lru_cache(maxsize=None)
def _get_sc_scatter():
  return pl.kernel(
    _sc_scatter_body,
    out_type=(
        jax.ShapeDtypeStruct((_NC, _ACC_ROWS, _BOND_LEN), jnp.float32),
        jax.ShapeDtypeStruct((_NC, _ACC_ROWS, _BOND_LEN), jnp.float32),
    ),
    mesh=plsc.VectorSubcoreMesh(core_axis_name="c", subcore_axis_name="s",
                                num_cores=_NC, num_subcores=_NS),
    compiler_params=pltpu.CompilerParams(use_tc_tiling_on_sc=False,
                                         needs_layout_passes=False),
    scratch_types=[
        pltpu.VMEM((_CHUNKS_PER_DMA, _CHUNK), jnp.int32),
        pltpu.VMEM((_CHUNKS_PER_DMA, _CHUNK), jnp.int32),
        pltpu.VMEM((_BOND_LEN, _DMA_EDGES), jnp.float32),
        pltpu.VMEM((_DMA_EDGES, _BOND_LEN), jnp.float32),
        pltpu.VMEM((_DMA_EDGES, _BOND_LEN), jnp.float32),
        pltpu.VMEM((_CHUNK, _BOND_LEN), jnp.float32),
        pltpu.VMEM((128, _BOND_LEN), jnp.float32),
        pltpu.SemaphoreType.DMA,
        pltpu.VMEM_SHARED((_ACC_ROWS, _BOND_LEN), jnp.float32),
        pltpu.VMEM_SHARED((_ACC_ROWS, _BOND_LEN), jnp.float32),
    ],
  )


def _tc_pre_body(sites_ref, gts_ref, states_ref, w1b_ref, w1c_ref, b1_ref,
                 h1p_ref):
    f32 = jnp.float32
    g = gts_ref[0, 0, :]
    iota = lax.broadcasted_iota(jnp.int32, (_ROW_TILE, _N_GRAPHS), 1)
    onehot = (g[:, None] == iota).astype(f32)
    sg = jnp.dot(onehot, states_ref[...], preferred_element_type=f32)
    h1p_ref[...] = (jnp.dot(sites_ref[...], w1b_ref[...],
                            preferred_element_type=f32)
                    + jnp.dot(sg, w1c_ref[...], preferred_element_type=f32)
                    + b1_ref[...])


_tc_pre = pl.pallas_call(
    _tc_pre_body,
    grid=(_N_TILES,),
    in_specs=[
        pl.BlockSpec((_ROW_TILE, _SITE_LEN), lambda i: (i, 0)),
        pl.BlockSpec((1, 1, _ROW_TILE), lambda i: (i, 0, 0)),
        pl.BlockSpec((_N_GRAPHS, _STATE_LEN), lambda i: (0, 0)),
        pl.BlockSpec((_SITE_LEN, _H1), lambda i: (0, 0)),
        pl.BlockSpec((_STATE_LEN, _H1), lambda i: (0, 0)),
        pl.BlockSpec((1, _H1), lambda i: (0, 0)),
    ],
    out_specs=pl.BlockSpec((_ROW_TILE, _H1), lambda i: (i, 0)),
    out_shape=jax.ShapeDtypeStruct((_N_SITES, _H1), jnp.float32),
)


def _tc_post_body(sum_ref, cnt_ref, h1p_ref, w1a_ref,
                  w2_ref, b2_ref, w3_ref, b3_ref, out_ref):
    f32 = jnp.float32
    seg_sum = sum_ref[0] + sum_ref[1]
    seg_cnt = cnt_ref[0] + cnt_ref[1]
    bp = seg_sum / jnp.maximum(seg_cnt, 1.0)

    h = jnp.maximum(h1p_ref[...]
                    + jnp.dot(bp, w1a_ref[...], preferred_element_type=f32),
                    0.0)
    h = jnp.maximum(jnp.dot(h, w2_ref[...], preferred_element_type=f32)
                    + b2_ref[...], 0.0)
    out_ref[...] = jnp.maximum(jnp.dot(h, w3_ref[...], preferred_element_type=f32)
                               + b3_ref[...], 0.0)


_POST_TILE = 2000

_tc_post = pl.pallas_call(
    _tc_post_body,
    grid=(_N_SITES // _POST_TILE,),
    in_specs=[
        pl.BlockSpec((_NC, _POST_TILE, _BOND_LEN), lambda i: (0, i, 0)),
        pl.BlockSpec((_NC, _POST_TILE, _BOND_LEN), lambda i: (0, i, 0)),
        pl.BlockSpec((_POST_TILE, _H1), lambda i: (i, 0)),
        pl.BlockSpec((_BOND_LEN, _H1), lambda i: (0, 0)),
        pl.BlockSpec((_H1, _H2), lambda i: (0, 0)),
        pl.BlockSpec((1, _H2), lambda i: (0, 0)),
        pl.BlockSpec((_H2, _SITE_LEN), lambda i: (0, 0)),
        pl.BlockSpec((1, _SITE_LEN), lambda i: (0, 0)),
    ],
    out_specs=pl.BlockSpec((_POST_TILE, _SITE_LEN), lambda i: (i, 0)),
    out_shape=jax.ShapeDtypeStruct((_N_SITES, _SITE_LEN), jnp.float32),
)


@jax.jit
def kernel(sites, bonds, states, W1, b1, W2, b2, W3, b3, indices1,
           graph_to_sites):
    i32 = jnp.int32

    idx_p = indices1.astype(i32).reshape(
        _NW, _EDGES_PER_WORKER // _CHUNK, _CHUNK)

    seg_sum, seg_cnt = _get_sc_scatter()(bonds.T, idx_p)

    gts_p = graph_to_sites.astype(i32).reshape(_N_TILES, 1, _ROW_TILE)

    h1p = _tc_pre(sites, gts_p, states,
                  W1[_BOND_LEN:_BOND_LEN + _SITE_LEN],
                  W1[_BOND_LEN + _SITE_LEN:], b1.reshape(1, _H1))
    out = _tc_post(seg_sum, seg_cnt, h1p, W1[:_BOND_LEN],
                   W2, b2.reshape(1, _H2), W3, b3.reshape(1, _SITE_LEN))
    return out


# speedup vs baseline: 1.2463x; 1.0199x over previous
import functools

import jax
import jax.numpy as jnp
from jax import lax
from jax.experimental import pallas as pl
from jax.experimental.pallas import tpu as pltpu
from jax.experimental.pallas import tpu_sc as plsc

_N_SITES = 10000
_N_EDGES = 320000
_N_GRAPHS = 64
_SITE_LEN = 256
_BOND_LEN = 16
_STATE_LEN = 128
_H1 = 512
_H2 = 512

_NC = 2
_NS = 16
_NW = _NC * _NS

_ROW_TILE = 400
_N_TILES = _N_SITES // _ROW_TILE
_ACC_ROWS = 10240
_ROWS_PER_TEC = _ACC_ROWS // _NS

_CNT_W = 8
_CHUNK = 125
_CHUNKS_PER_DMA = 16
_DMA_EDGES = _CHUNK * _CHUNKS_PER_DMA
_EDGES_PER_WORKER = _N_EDGES // _NW
_DMAS_PER_WORKER = _EDGES_PER_WORKER // _DMA_EDGES
_GROUPS = _DMA_EDGES // 16


def _sc_scatter_body(bt_hbm, idx_hbm, sum_hbm, cnt_hbm,
                     idx_va, idx_vb, xbuf, rowbuf_a, rowbuf_b, ones_v, zbuf,
                     zcnt, scat_sem, acc_sum, acc_cnt):
    cid = lax.axis_index("c")
    tid = lax.axis_index("s")
    wid = tid * _NC + cid
    i32 = jnp.int32
    f32 = jnp.float32

    def _zero(i, _):
        zbuf[i] = jnp.zeros((16,), f32)
        return 0
    lax.fori_loop(0, 128, _zero, 0)

    iota16f = lax.iota(i32, 16)
    for gch in range(8):
        rr = iota16f + gch * 16
        for c in range(_CNT_W):
            cc = jnp.full((16,), c, i32)
            plsc.store_scatter(ones_v, [rr, cc], jnp.ones((16,), f32))
            plsc.store_scatter(zcnt, [rr, cc], jnp.zeros((16,), f32))

    for k in range(_ROWS_PER_TEC // 128):
        sl0 = pl.ds(tid * _ROWS_PER_TEC + k * 128, 128)
        pltpu.sync_copy(zbuf, acc_sum.at[sl0])
        pltpu.sync_copy(zcnt, acc_cnt.at[sl0])
    plsc.subcore_barrier()

    base = wid * _EDGES_PER_WORKER
    iota16 = lax.iota(i32, 16)
    cols = [jnp.full((16,), f, i32) for f in range(_BOND_LEN)]

    descs = {}
    for o in range(_DMAS_PER_WORKER):
        idx_v = idx_va if o % 2 == 0 else idx_vb
        rowbuf = rowbuf_a if o % 2 == 0 else rowbuf_b
        if o >= 2:
            for d in descs.pop(o - 2):
                d.wait()

        pltpu.sync_copy(idx_hbm.at[wid, pl.ds(o * _CHUNKS_PER_DMA, _CHUNKS_PER_DMA)],
                        idx_v)
        pltpu.sync_copy(bt_hbm.at[:, pl.ds(base + o * _DMA_EDGES, _DMA_EDGES)],
                        xbuf)

        def _tr(g, _, xbuf=xbuf, rowbuf=rowbuf):
            rows = iota16 + g * 16
            for f in range(_BOND_LEN):
                vals = xbuf[f, pl.ds(g * 16, 16)]
                plsc.store_scatter(rowbuf, [rows, cols[f]], vals)
            return 0
        lax.fori_loop(0, _GROUPS, _tr, 0)

        ds = []
        for j in range(_CHUNKS_PER_DMA):
            idx_row = idx_v.at[j]
            ds.append(pltpu.async_copy(rowbuf.at[pl.ds(j * _CHUNK, _CHUNK)],
                                       acc_sum.at[idx_row], scat_sem, add=True))
            ds.append(pltpu.async_copy(ones_v.at[pl.ds(0, _CHUNK)],
                                       acc_cnt.at[idx_row], scat_sem,
                                       add=True))
        descs[o] = ds
    for o in sorted(descs):
        for d in descs[o]:
            d.wait()

    plsc.subcore_barrier()
    sl = pl.ds(tid * _ROWS_PER_TEC, _ROWS_PER_TEC)
    pltpu.sync_copy(acc_sum.at[sl], sum_hbm.at[cid, sl])
    pltpu.sync_copy(acc_cnt.at[sl], cnt_hbm.at[cid, sl])


@functools.lru_cache(maxsize=None)
def _get_sc_scatter():
  return pl.kernel(
    _sc_scatter_body,
    out_type=(
        jax.ShapeDtypeStruct((_NC, _ACC_ROWS, _BOND_LEN), jnp.float32),
        jax.ShapeDtypeStruct((_NC, _ACC_ROWS, _CNT_W), jnp.float32),
    ),
    mesh=plsc.VectorSubcoreMesh(core_axis_name="c", subcore_axis_name="s",
                                num_cores=_NC, num_subcores=_NS),
    compiler_params=pltpu.CompilerParams(use_tc_tiling_on_sc=False,
                                         needs_layout_passes=False),
    scratch_types=[
        pltpu.VMEM((_CHUNKS_PER_DMA, _CHUNK), jnp.int32),
        pltpu.VMEM((_CHUNKS_PER_DMA, _CHUNK), jnp.int32),
        pltpu.VMEM((_BOND_LEN, _DMA_EDGES), jnp.float32),
        pltpu.VMEM((_DMA_EDGES, _BOND_LEN), jnp.float32),
        pltpu.VMEM((_DMA_EDGES, _BOND_LEN), jnp.float32),
        pltpu.VMEM((128, _CNT_W), jnp.float32),
        pltpu.VMEM((128, _BOND_LEN), jnp.float32),
        pltpu.VMEM((128, _CNT_W), jnp.float32),
        pltpu.SemaphoreType.DMA,
        pltpu.VMEM_SHARED((_ACC_ROWS, _BOND_LEN), jnp.float32),
        pltpu.VMEM_SHARED((_ACC_ROWS, _CNT_W), jnp.float32),
    ],
  )


def _tc_pre_body(sites_ref, gts_ref, states_ref, w1b_ref, w1c_ref, b1_ref,
                 h1p_ref):
    f32 = jnp.float32
    g = gts_ref[0, 0, :]
    iota = lax.broadcasted_iota(jnp.int32, (_ROW_TILE, _N_GRAPHS), 1)
    onehot = (g[:, None] == iota).astype(f32)
    sg = jnp.dot(onehot, states_ref[...], preferred_element_type=f32)
    h1p_ref[...] = (jnp.dot(sites_ref[...], w1b_ref[...],
                            preferred_element_type=f32)
                    + jnp.dot(sg, w1c_ref[...], preferred_element_type=f32)
                    + b1_ref[...])


_tc_pre = pl.pallas_call(
    _tc_pre_body,
    grid=(_N_TILES,),
    in_specs=[
        pl.BlockSpec((_ROW_TILE, _SITE_LEN), lambda i: (i, 0)),
        pl.BlockSpec((1, 1, _ROW_TILE), lambda i: (i, 0, 0)),
        pl.BlockSpec((_N_GRAPHS, _STATE_LEN), lambda i: (0, 0)),
        pl.BlockSpec((_SITE_LEN, _H1), lambda i: (0, 0)),
        pl.BlockSpec((_STATE_LEN, _H1), lambda i: (0, 0)),
        pl.BlockSpec((1, _H1), lambda i: (0, 0)),
    ],
    out_specs=pl.BlockSpec((_ROW_TILE, _H1), lambda i: (i, 0)),
    out_shape=jax.ShapeDtypeStruct((_N_SITES, _H1), jnp.float32),
)


def _tc_post_body(sum_ref, cnt_ref, h1p_ref, w1a_ref,
                  w2_ref, b2_ref, w3_ref, b3_ref, out_ref):
    f32 = jnp.float32
    seg_sum = sum_ref[0] + sum_ref[1]
    seg_cnt = cnt_ref[0, :, :1] + cnt_ref[1, :, :1]
    bp = seg_sum / jnp.maximum(seg_cnt, 1.0)

    h = jnp.maximum(h1p_ref[...]
                    + jnp.dot(bp, w1a_ref[...], preferred_element_type=f32),
                    0.0)
    h = jnp.maximum(jnp.dot(h, w2_ref[...], preferred_element_type=f32)
                    + b2_ref[...], 0.0)
    out_ref[...] = jnp.maximum(jnp.dot(h, w3_ref[...], preferred_element_type=f32)
                               + b3_ref[...], 0.0)


_POST_TILE = 2000

_tc_post = pl.pallas_call(
    _tc_post_body,
    grid=(_N_SITES // _POST_TILE,),
    in_specs=[
        pl.BlockSpec((_NC, _POST_TILE, _BOND_LEN), lambda i: (0, i, 0)),
        pl.BlockSpec((_NC, _POST_TILE, _CNT_W), lambda i: (0, i, 0)),
        pl.BlockSpec((_POST_TILE, _H1), lambda i: (i, 0)),
        pl.BlockSpec((_BOND_LEN, _H1), lambda i: (0, 0)),
        pl.BlockSpec((_H1, _H2), lambda i: (0, 0)),
        pl.BlockSpec((1, _H2), lambda i: (0, 0)),
        pl.BlockSpec((_H2, _SITE_LEN), lambda i: (0, 0)),
        pl.BlockSpec((1, _SITE_LEN), lambda i: (0, 0)),
    ],
    out_specs=pl.BlockSpec((_POST_TILE, _SITE_LEN), lambda i: (i, 0)),
    out_shape=jax.ShapeDtypeStruct((_N_SITES, _SITE_LEN), jnp.float32),
)


@jax.jit
def kernel(sites, bonds, states, W1, b1, W2, b2, W3, b3, indices1,
           graph_to_sites):
    i32 = jnp.int32

    idx_p = indices1.astype(i32).reshape(
        _NW, _EDGES_PER_WORKER // _CHUNK, _CHUNK)

    seg_sum, seg_cnt = _get_sc_scatter()(bonds.T, idx_p)

    gts_p = graph_to_sites.astype(i32).reshape(_N_TILES, 1, _ROW_TILE)

    h1p = _tc_pre(sites, gts_p, states,
                  W1[_BOND_LEN:_BOND_LEN + _SITE_LEN],
                  W1[_BOND_LEN + _SITE_LEN:], b1.reshape(1, _H1))
    out = _tc_post(seg_sum, seg_cnt, h1p, W1[:_BOND_LEN],
                   W2, b2.reshape(1, _H2), W3, b3.reshape(1, _SITE_LEN))
    return out


# speedup vs baseline: 1.2960x; 1.0399x over previous
import functools

import jax
import jax.numpy as jnp
from jax import lax
from jax.experimental import pallas as pl
from jax.experimental.pallas import tpu as pltpu
from jax.experimental.pallas import tpu_sc as plsc

_N_SITES = 10000
_N_EDGES = 320000
_N_GRAPHS = 64
_SITE_LEN = 256
_BOND_LEN = 16
_STATE_LEN = 128
_H1 = 512
_H2 = 512

_NC = 2
_NS = 16
_NW = _NC * _NS

_ROW_TILE = 400
_N_TILES = _N_SITES // _ROW_TILE
_ACC_ROWS = 10240
_ROWS_PER_TEC = _ACC_ROWS // _NS

_CNT_W = 8
_CHUNK = 125
_CHUNKS_PER_DMA = 16
_DMA_EDGES = _CHUNK * _CHUNKS_PER_DMA
_EDGES_PER_WORKER = _N_EDGES // _NW
_DMAS_PER_WORKER = _EDGES_PER_WORKER // _DMA_EDGES
_GROUPS = _DMA_EDGES // 16


def _sc_scatter_body(bt_hbm, idx_hbm, sum_hbm, cnt_hbm,
                     idx_va, idx_vb, xbuf, rowbuf_a, rowbuf_b, ones_v, zbuf,
                     zcnt, scat_sem, acc_sum, acc_cnt):
    cid = lax.axis_index("c")
    tid = lax.axis_index("s")
    wid = tid * _NC + cid
    i32 = jnp.int32
    f32 = jnp.float32

    def _zero(i, _):
        zbuf[i] = jnp.zeros((16,), f32)
        return 0
    lax.fori_loop(0, 128, _zero, 0)

    iota16f = lax.iota(i32, 16)
    for gch in range(8):
        rr = iota16f + gch * 16
        for c in range(_CNT_W):
            cc = jnp.full((16,), c, i32)
            plsc.store_scatter(ones_v, [rr, cc], jnp.ones((16,), f32))
            plsc.store_scatter(zcnt, [rr, cc], jnp.zeros((16,), f32))

    for k in range(_ROWS_PER_TEC // 128):
        sl0 = pl.ds(tid * _ROWS_PER_TEC + k * 128, 128)
        pltpu.sync_copy(zbuf, acc_sum.at[sl0])
        pltpu.sync_copy(zcnt, acc_cnt.at[sl0])
    plsc.subcore_barrier()

    base = wid * _EDGES_PER_WORKER
    iota16 = lax.iota(i32, 16)
    cols = [jnp.full((16,), f, i32) for f in range(_BOND_LEN)]

    descs = {}
    for o in range(_DMAS_PER_WORKER):
        idx_v = idx_va if o % 2 == 0 else idx_vb
        rowbuf = rowbuf_a if o % 2 == 0 else rowbuf_b
        if o >= 2:
            for d in descs.pop(o - 2):
                d.wait()

        pltpu.sync_copy(idx_hbm.at[wid, pl.ds(o * _CHUNKS_PER_DMA, _CHUNKS_PER_DMA)],
                        idx_v)
        pltpu.sync_copy(bt_hbm.at[:, pl.ds(base + o * _DMA_EDGES, _DMA_EDGES)],
                        xbuf)

        @plsc.parallel_loop(0, _GROUPS, unroll=4)
        def _tr(g, xbuf=xbuf, rowbuf=rowbuf):
            rows = iota16 + g * 16
            for f in range(_BOND_LEN):
                vals = xbuf[f, pl.ds(g * 16, 16)]
                plsc.store_scatter(rowbuf, [rows, cols[f]], vals)

        ds = []
        for j in range(_CHUNKS_PER_DMA):
            idx_row = idx_v.at[j]
            ds.append(pltpu.async_copy(rowbuf.at[pl.ds(j * _CHUNK, _CHUNK)],
                                       acc_sum.at[idx_row], scat_sem, add=True))
            ds.append(pltpu.async_copy(ones_v.at[pl.ds(0, _CHUNK)],
                                       acc_cnt.at[idx_row], scat_sem,
                                       add=True))
        descs[o] = ds
    for o in sorted(descs):
        for d in descs[o]:
            d.wait()

    plsc.subcore_barrier()
    sl = pl.ds(tid * _ROWS_PER_TEC, _ROWS_PER_TEC)
    pltpu.sync_copy(acc_sum.at[sl], sum_hbm.at[cid, sl])
    pltpu.sync_copy(acc_cnt.at[sl], cnt_hbm.at[cid, sl])


@functools.lru_cache(maxsize=None)
def _get_sc_scatter():
  return pl.kernel(
    _sc_scatter_body,
    out_type=(
        jax.ShapeDtypeStruct((_NC, _ACC_ROWS, _BOND_LEN), jnp.float32),
        jax.ShapeDtypeStruct((_NC, _ACC_ROWS, _CNT_W), jnp.float32),
    ),
    mesh=plsc.VectorSubcoreMesh(core_axis_name="c", subcore_axis_name="s",
                                num_cores=_NC, num_subcores=_NS),
    compiler_params=pltpu.CompilerParams(use_tc_tiling_on_sc=False,
                                         needs_layout_passes=False),
    scratch_types=[
        pltpu.VMEM((_CHUNKS_PER_DMA, _CHUNK), jnp.int32),
        pltpu.VMEM((_CHUNKS_PER_DMA, _CHUNK), jnp.int32),
        pltpu.VMEM((_BOND_LEN, _DMA_EDGES), jnp.float32),
        pltpu.VMEM((_DMA_EDGES, _BOND_LEN), jnp.float32),
        pltpu.VMEM((_DMA_EDGES, _BOND_LEN), jnp.float32),
        pltpu.VMEM((128, _CNT_W), jnp.float32),
        pltpu.VMEM((128, _BOND_LEN), jnp.float32),
        pltpu.VMEM((128, _CNT_W), jnp.float32),
        pltpu.SemaphoreType.DMA,
        pltpu.VMEM_SHARED((_ACC_ROWS, _BOND_LEN), jnp.float32),
        pltpu.VMEM_SHARED((_ACC_ROWS, _CNT_W), jnp.float32),
    ],
  )


def _tc_pre_body(sites_ref, gts_ref, states_ref, w1b_ref, w1c_ref, b1_ref,
                 h1p_ref):
    f32 = jnp.float32
    g = gts_ref[0, 0, :]
    iota = lax.broadcasted_iota(jnp.int32, (_ROW_TILE, _N_GRAPHS), 1)
    onehot = (g[:, None] == iota).astype(f32)
    sg = jnp.dot(onehot, states_ref[...], preferred_element_type=f32)
    h1p_ref[...] = (jnp.dot(sites_ref[...], w1b_ref[...],
                            preferred_element_type=f32)
                    + jnp.dot(sg, w1c_ref[...], preferred_element_type=f32)
                    + b1_ref[...])


_tc_pre = pl.pallas_call(
    _tc_pre_body,
    grid=(_N_TILES,),
    in_specs=[
        pl.BlockSpec((_ROW_TILE, _SITE_LEN), lambda i: (i, 0)),
        pl.BlockSpec((1, 1, _ROW_TILE), lambda i: (i, 0, 0)),
        pl.BlockSpec((_N_GRAPHS, _STATE_LEN), lambda i: (0, 0)),
        pl.BlockSpec((_SITE_LEN, _H1), lambda i: (0, 0)),
        pl.BlockSpec((_STATE_LEN, _H1), lambda i: (0, 0)),
        pl.BlockSpec((1, _H1), lambda i: (0, 0)),
    ],
    out_specs=pl.BlockSpec((_ROW_TILE, _H1), lambda i: (i, 0)),
    out_shape=jax.ShapeDtypeStruct((_N_SITES, _H1), jnp.float32),
)


def _tc_post_body(sum_ref, cnt_ref, h1p_ref, w1a_ref,
                  w2_ref, b2_ref, w3_ref, b3_ref, out_ref):
    f32 = jnp.float32
    seg_sum = sum_ref[0] + sum_ref[1]
    seg_cnt = cnt_ref[0, :, :1] + cnt_ref[1, :, :1]
    bp = seg_sum / jnp.maximum(seg_cnt, 1.0)

    h = jnp.maximum(h1p_ref[...]
                    + jnp.dot(bp, w1a_ref[...], preferred_element_type=f32),
                    0.0)
    h = jnp.maximum(jnp.dot(h, w2_ref[...], preferred_element_type=f32)
                    + b2_ref[...], 0.0)
    out_ref[...] = jnp.maximum(jnp.dot(h, w3_ref[...], preferred_element_type=f32)
                               + b3_ref[...], 0.0)


_POST_TILE = 2000

_tc_post = pl.pallas_call(
    _tc_post_body,
    grid=(_N_SITES // _POST_TILE,),
    in_specs=[
        pl.BlockSpec((_NC, _POST_TILE, _BOND_LEN), lambda i: (0, i, 0)),
        pl.BlockSpec((_NC, _POST_TILE, _CNT_W), lambda i: (0, i, 0)),
        pl.BlockSpec((_POST_TILE, _H1), lambda i: (i, 0)),
        pl.BlockSpec((_BOND_LEN, _H1), lambda i: (0, 0)),
        pl.BlockSpec((_H1, _H2), lambda i: (0, 0)),
        pl.BlockSpec((1, _H2), lambda i: (0, 0)),
        pl.BlockSpec((_H2, _SITE_LEN), lambda i: (0, 0)),
        pl.BlockSpec((1, _SITE_LEN), lambda i: (0, 0)),
    ],
    out_specs=pl.BlockSpec((_POST_TILE, _SITE_LEN), lambda i: (i, 0)),
    out_shape=jax.ShapeDtypeStruct((_N_SITES, _SITE_LEN), jnp.float32),
)


@jax.jit
def kernel(sites, bonds, states, W1, b1, W2, b2, W3, b3, indices1,
           graph_to_sites):
    i32 = jnp.int32

    idx_p = indices1.astype(i32).reshape(
        _NW, _EDGES_PER_WORKER // _CHUNK, _CHUNK)

    seg_sum, seg_cnt = _get_sc_scatter()(bonds.T, idx_p)

    gts_p = graph_to_sites.astype(i32).reshape(_N_TILES, 1, _ROW_TILE)

    h1p = _tc_pre(sites, gts_p, states,
                  W1[_BOND_LEN:_BOND_LEN + _SITE_LEN],
                  W1[_BOND_LEN + _SITE_LEN:], b1.reshape(1, _H1))
    out = _tc_post(seg_sum, seg_cnt, h1p, W1[:_BOND_LEN],
                   W2, b2.reshape(1, _H2), W3, b3.reshape(1, _SITE_LEN))
    return out
